# baseline (speedup 1.0000x reference)
import functools

import jax
import jax.numpy as jnp
import numpy as np
from jax import lax
from jax.experimental import pallas as pl
from jax.experimental.pallas import tpu as pltpu
from jax.experimental.pallas import tpu_sc as plsc

N_SCALARS = 8
NUM_REP = 4
L_MAX = 2
ROW = 24
NB_E = 4
LW = ROW * NB_E
DIST = 16
HID = 72

_MESH = dict(core_axis_name="c", subcore_axis_name="s", num_cores=2,
             num_subcores=16)
NW = 32
IDXB = 80
CH = 800
NB = CH // IDXB


def _sc_gather(table, idx, n_edges):
    per_w = n_edges // NW
    n_chunks = per_w // CH

    @functools.partial(
        pl.kernel,
        mesh=plsc.VectorSubcoreMesh(**_MESH),
        compiler_params=pltpu.CompilerParams(use_tc_tiling_on_sc=False),
        out_type=jax.ShapeDtypeStruct((n_edges, ROW), jnp.float32),
        scratch_types=[
            pltpu.VMEM((CH,), jnp.int32),
            pltpu.VMEM((CH, ROW), jnp.float32),
            pltpu.SemaphoreType.DMA,
        ],
    )
    def gk(table_hbm, idx_hbm, out_hbm, idx_v, rows_v, sem):
        wid = lax.axis_index("s") * 2 + lax.axis_index("c")
        base = wid * per_w

        def body(i, carry):
            start = pl.multiple_of(base + i * CH, CH)
            pltpu.sync_copy(idx_hbm.at[pl.ds(start, CH)], idx_v)
            handles = []
            for b in range(NB):
                handles.append(pltpu.async_copy(
                    table_hbm.at[idx_v.at[pl.ds(b * IDXB, IDXB)]],
                    rows_v.at[pl.ds(b * IDXB, IDXB)], sem))
            for h in handles:
                h.wait()
            pltpu.sync_copy(rows_v, out_hbm.at[pl.ds(start, CH)])
            return carry

        lax.fori_loop(0, n_chunks, body, 0)

    return gk(table, idx)


def _make_consts():
    pa = np.zeros((32, LW), np.float32)
    pb = np.zeros((32, LW), np.float32)
    pc = np.zeros((32, LW), np.float32)
    for e in range(NB_E):
        for p in range(8):
            k = p >> 2
            pa[e * 8 + k * 4 + 0, e * ROW + 8 + p] = 1.0
            pa[e * 8 + k * 4 + 3, e * ROW + 16 + p] = 1.0
            pb[e * 8 + k * 4 + 1, e * ROW + 8 + p] = 1.0
            pc[e * 8 + k * 4 + 2, e * ROW + 16 + p] = 1.0
    ones8 = np.zeros((LW,), np.float32)
    for e in range(NB_E):
        ones8[e * ROW:e * ROW + 8] = 1.0
    return pa, pb, pc, ones8


_PA, _PB, _PC, _ONES8 = _make_consts()


def _dense_body(g_ref, de_ref, rot_ref, pa_ref, pb_ref, pc_ref, ones_ref,
                w1_ref, b1_ref, w2_ref, b2_ref, out_ref):
    f32 = jnp.float32
    rot_b = rot_ref[...]
    a = jnp.dot(rot_b, pa_ref[...], preferred_element_type=f32) + ones_ref[...]
    bco = jnp.dot(rot_b, pb_ref[...], preferred_element_type=f32)
    cco = jnp.dot(rot_b, pc_ref[...], preferred_element_type=f32)

    g = g_ref[...]
    u = (g * a + pltpu.roll(g, LW - 8, 1) * bco
         + pltpu.roll(g, 8, 1) * cco)

    x_in = jnp.concatenate([de_ref[...], u], axis=1)
    h = jnp.dot(x_in, w1_ref[...], preferred_element_type=f32) + b1_ref[...]
    h = jnp.maximum(h, 0.0)
    o = jnp.dot(h, w2_ref[...], preferred_element_type=f32) + b2_ref[...]

    bo = pltpu.roll(cco, LW - 8, 1)
    co = pltpu.roll(bco, 8, 1)
    out_ref[...] = (o * a + pltpu.roll(o, LW - 8, 1) * bo
                    + pltpu.roll(o, 8, 1) * co)


def _dense_stage(gathered, de, rot_flat, w1bd, b1t, w2bd, b2t, blk=1000):
    n_rows = gathered.shape[0] // NB_E
    g96 = gathered.reshape(n_rows, LW)
    de64 = de.reshape(n_rows, NB_E * DIST)
    rot32 = rot_flat.reshape(n_rows, NB_E * 8)
    out = pl.pallas_call(
        _dense_body,
        grid=(n_rows // blk,),
        in_specs=[
            pl.BlockSpec((blk, LW), lambda i: (i, 0)),
            pl.BlockSpec((blk, NB_E * DIST), lambda i: (i, 0)),
            pl.BlockSpec((blk, NB_E * 8), lambda i: (i, 0)),
            pl.BlockSpec((NB_E * 8, LW), lambda i: (0, 0)),
            pl.BlockSpec((NB_E * 8, LW), lambda i: (0, 0)),
            pl.BlockSpec((NB_E * 8, LW), lambda i: (0, 0)),
            pl.BlockSpec((1, LW), lambda i: (0, 0)),
            pl.BlockSpec((NB_E * DIST + LW, NB_E * HID), lambda i: (0, 0)),
            pl.BlockSpec((1, NB_E * HID), lambda i: (0, 0)),
            pl.BlockSpec((NB_E * HID, LW), lambda i: (0, 0)),
            pl.BlockSpec((1, LW), lambda i: (0, 0)),
        ],
        out_specs=pl.BlockSpec((blk, LW), lambda i: (i, 0)),
        out_shape=jax.ShapeDtypeStruct((n_rows, LW), jnp.float32),
    )(g96, de64, rot32, jnp.asarray(_PA), jnp.asarray(_PB),
      jnp.asarray(_PC), jnp.asarray(_ONES8).reshape(1, LW),
      w1bd, b1t, w2bd, b2t)
    return out.reshape(n_rows * NB_E, ROW)


def _make_weights(W1, b1, W2, b2):
    yrow = lambda p, l: 24 + (p & 3) * 4 + (p >> 2) * 2 + l
    in_rows = ([16 + r for r in range(8)]
               + [yrow(p, 0) for p in range(8)]
               + [yrow(p, 1) for p in range(8)])
    mcol = lambda p, m: 8 + (p & 3) * 4 + (p >> 2) * 2 + m
    out_cols = (list(range(8)) + [mcol(p, 0) for p in range(8)]
                + [mcol(p, 1) for p in range(8)])
    out_cols = np.array(out_cols)

    w1bd = jnp.zeros((NB_E * DIST + LW, NB_E * HID), jnp.float32)
    w2bd = jnp.zeros((NB_E * HID, LW), jnp.float32)
    for e in range(NB_E):
        w1bd = w1bd.at[e * DIST:(e + 1) * DIST,
                       e * HID:(e + 1) * HID].set(W1[:DIST])
        w1bd = w1bd.at[NB_E * DIST + e * ROW:NB_E * DIST + (e + 1) * ROW,
                       e * HID:(e + 1) * HID].set(W1[np.array(in_rows)])
        w2bd = w2bd.at[e * HID:(e + 1) * HID,
                       e * ROW:(e + 1) * ROW].set(W2[:, out_cols])
    b1t = jnp.tile(b1, NB_E).reshape(1, NB_E * HID)
    b2t = jnp.tile(b2[out_cols], NB_E).reshape(1, LW)
    return w1bd, b1t, w2bd, b2t


def _sc_scatter(msgs, col, n_edges, n_nodes):
    half = n_nodes // 2
    acc_rows = 51200
    per_tec = n_edges // 16
    n_chunks = per_tec // CH
    zero_rows_per_tec = acc_rows // 16
    drain_a = 3128
    drain_b = half - 15 * drain_a

    @functools.partial(
        pl.kernel,
        mesh=plsc.VectorSubcoreMesh(**_MESH),
        compiler_params=pltpu.CompilerParams(use_tc_tiling_on_sc=False),
        out_type=jax.ShapeDtypeStruct((n_nodes, ROW), jnp.float32),
        scratch_types=[
            pltpu.VMEM((2, CH), jnp.int32),
            pltpu.VMEM((NB, IDXB), jnp.int32),
            pltpu.VMEM((2 * CH, ROW), jnp.float32),
            pltpu.VMEM_SHARED((acc_rows, ROW), jnp.float32),
            pltpu.SemaphoreType.DMA,
            pltpu.SemaphoreType.DMA,
            pltpu.SemaphoreType.DMA,
        ],
    )
    def sk(msg_hbm, col_hbm, out_hbm, col_v, loc_v, msg_v, acc,
           lsem0, lsem1, ssem):
        c = lax.axis_index("c")
        t = lax.axis_index("s")
        node_base = c * half

        zeros16 = jnp.zeros((16,), jnp.float32)

        def zbody(i, carry):
            msg_v[i // 2, pl.ds((i % 2) * 16 - (i % 2) * 8, 16)] = zeros16
            return carry
        lax.fori_loop(0, CH * 2, zbody, 0)
        for q in range(zero_rows_per_tec // CH):
            pltpu.sync_copy(
                msg_v.at[pl.ds(0, CH)],
                acc.at[pl.ds(t * zero_rows_per_tec + q * CH, CH)])
        plsc.subcore_barrier()

        def chunk_start(ci):
            return pl.multiple_of(t * per_tec + ci * CH, CH)

        def issue_loads(ci, buf, lsem):
            pltpu.async_copy(col_hbm.at[pl.ds(chunk_start(ci), CH)],
                             col_v.at[buf], lsem)
            pltpu.async_copy(msg_hbm.at[pl.ds(chunk_start(ci), CH)],
                             msg_v.at[pl.ds(buf * CH, CH)], lsem)

        def wait_loads(ci, buf, lsem):
            pltpu.make_async_copy(col_hbm.at[pl.ds(chunk_start(ci), CH)],
                                  col_v.at[buf], lsem).wait()
            pltpu.make_async_copy(msg_hbm.at[pl.ds(chunk_start(ci), CH)],
                                  msg_v.at[pl.ds(buf * CH, CH)], lsem).wait()

        def process(ci, buf):
            for b in range(NB):
                for gs in range(IDXB // 16):
                    v = col_v[buf, pl.ds(b * IDXB + gs * 16, 16)]
                    li = v - node_base
                    ok = (li >= 0) & (li < half)
                    loc_v[b, pl.ds(gs * 16, 16)] = jnp.where(ok, li, half)
            handles = []
            for b in range(NB):
                handles.append(pltpu.async_copy(
                    msg_v.at[pl.ds(buf * CH + b * IDXB, IDXB)],
                    acc.at[loc_v.at[b]], ssem, add=True))
            for h in handles:
                h.wait()

        issue_loads(0, 0, lsem0)

        def body(i, carry):
            ci0 = i * 2
            wait_loads(ci0, 0, lsem0)

            @pl.when(ci0 + 1 < n_chunks)
            def _():
                issue_loads(ci0 + 1, 1, lsem1)
            process(ci0, 0)

            @pl.when(ci0 + 1 < n_chunks)
            def _():
                wait_loads(ci0 + 1, 1, lsem1)

                @pl.when(ci0 + 2 < n_chunks)
                def _():
                    issue_loads(ci0 + 2, 0, lsem0)
                process(ci0 + 1, 1)
            return carry

        lax.fori_loop(0, (n_chunks + 1) // 2, body, 0)
        plsc.subcore_barrier()
        off = pl.multiple_of(t * drain_a, 8)

        @pl.when(t < 15)
        def _():
            pltpu.sync_copy(
                acc.at[pl.ds(off, drain_a)],
                out_hbm.at[pl.ds(pl.multiple_of(node_base + off, 8),
                                 drain_a)])

        @pl.when(t == 15)
        def _():
            pltpu.sync_copy(
                acc.at[pl.ds(15 * drain_a, drain_b)],
                out_hbm.at[pl.ds(pl.multiple_of(node_base + 15 * drain_a, 8),
                                 drain_b)])

    return sk(msgs, col)


def kernel(x_scalar, x_rot, edge_index, distance_embedding, rot, W1, b1, W2,
           b2):
    n_nodes = x_rot.shape[0]
    n_edges = edge_index.shape[1]

    row = edge_index[0].astype(jnp.int32)
    col = edge_index[1].astype(jnp.int32)

    xr = x_rot.reshape(n_nodes, NUM_REP, L_MAX, 2)
    xr_m = jnp.transpose(xr, (0, 3, 2, 1)).reshape(n_nodes, 16)
    table = jnp.concatenate([x_scalar, xr_m], axis=1)

    w1bd, b1t, w2bd, b2t = _make_weights(W1, b1, W2, b2)

    gathered = _sc_gather(table, row, n_edges)
    msgs = _dense_stage(gathered, distance_embedding,
                        rot.reshape(n_edges, 8), w1bd, b1t, w2bd, b2t)
    acc = _sc_scatter(msgs, col, n_edges, n_nodes)

    mess_scalar = acc[:, :N_SCALARS]
    zm = acc[:, 8:24].reshape(n_nodes, 2, 2, NUM_REP)
    mess_rot = jnp.transpose(zm, (0, 3, 2, 1)).reshape(
        n_nodes, NUM_REP, L_MAX * 2)
    return (mess_scalar, mess_rot)


# speedup vs baseline: 4.4655x; 4.4655x over previous
import functools

import jax
import jax.numpy as jnp
import numpy as np
from jax import lax
from jax.experimental import pallas as pl
from jax.experimental.pallas import tpu as pltpu
from jax.experimental.pallas import tpu_sc as plsc

N_SCALARS = 8
NUM_REP = 4
L_MAX = 2
ROW = 24
NB_E = 4
LW = ROW * NB_E
DIST = 16
HID = 72

_MESH = dict(core_axis_name="c", subcore_axis_name="s", num_cores=2,
             num_subcores=16)
NW = 32
IDXB = 80
CR = 200
CHE = CR * NB_E
NBURST = CHE // IDXB


def _sc_gather(table, idx, n_edges):
    q = n_edges // NB_E
    rows_per_w = q // NW
    n_chunks = rows_per_w // CR

    @functools.partial(
        pl.kernel,
        mesh=plsc.VectorSubcoreMesh(**_MESH),
        compiler_params=pltpu.CompilerParams(use_tc_tiling_on_sc=False),
        out_type=jax.ShapeDtypeStruct((q, LW), jnp.float32),
        scratch_types=[
            pltpu.VMEM((CHE,), jnp.int32),
            pltpu.VMEM((CHE, ROW), jnp.float32),
            pltpu.SemaphoreType.DMA,
        ],
    )
    def gk(table_hbm, idx_hbm, out_hbm, idx_v, rows_v, sem):
        wid = lax.axis_index("s") * 2 + lax.axis_index("c")
        base = wid * rows_per_w

        def body(i, carry):
            row0 = pl.multiple_of(base + i * CR, CR)
            for j in range(NB_E):
                pltpu.sync_copy(idx_hbm.at[pl.ds(j * q + row0, CR)],
                                idx_v.at[pl.ds(j * CR, CR)])
            handles = []
            for b in range(NBURST):
                handles.append(pltpu.async_copy(
                    table_hbm.at[idx_v.at[pl.ds(b * IDXB, IDXB)]],
                    rows_v.at[pl.ds(b * IDXB, IDXB)], sem))
            for h in handles:
                h.wait()
            for j in range(NB_E):
                pltpu.sync_copy(
                    rows_v.at[pl.ds(j * CR, CR)],
                    out_hbm.at[pl.ds(row0, CR), pl.ds(j * ROW, ROW)])
            return carry

        lax.fori_loop(0, n_chunks, body, 0)

    return gk(table, idx)


def _make_consts():
    pa = np.zeros((32, LW), np.float32)
    pb = np.zeros((32, LW), np.float32)
    pc = np.zeros((32, LW), np.float32)
    for e in range(NB_E):
        for p in range(8):
            k = p >> 2
            pa[e * 8 + k * 4 + 0, e * ROW + 8 + p] = 1.0
            pa[e * 8 + k * 4 + 3, e * ROW + 16 + p] = 1.0
            pb[e * 8 + k * 4 + 1, e * ROW + 8 + p] = 1.0
            pc[e * 8 + k * 4 + 2, e * ROW + 16 + p] = 1.0
    ones8 = np.zeros((LW,), np.float32)
    for e in range(NB_E):
        ones8[e * ROW:e * ROW + 8] = 1.0
    return pa, pb, pc, ones8


_PA, _PB, _PC, _ONES8 = _make_consts()


def _dense_body(g_ref, de0_ref, de1_ref, de2_ref, de3_ref,
                rot0_ref, rot1_ref, rot2_ref, rot3_ref,
                pa_ref, pb_ref, pc_ref, ones_ref,
                w1_ref, b1_ref, w2_ref, b2_ref, out_ref):
    f32 = jnp.float32
    rot_b = jnp.concatenate(
        [rot0_ref[...], rot1_ref[...], rot2_ref[...], rot3_ref[...]],
        axis=1)
    a = jnp.dot(rot_b, pa_ref[...], preferred_element_type=f32) + ones_ref[...]
    bco = jnp.dot(rot_b, pb_ref[...], preferred_element_type=f32)
    cco = jnp.dot(rot_b, pc_ref[...], preferred_element_type=f32)

    g = g_ref[...]
    u = (g * a + pltpu.roll(g, LW - 8, 1) * bco
         + pltpu.roll(g, 8, 1) * cco)

    x_in = jnp.concatenate(
        [de0_ref[...], de1_ref[...], de2_ref[...], de3_ref[...], u],
        axis=1)
    h = jnp.dot(x_in, w1_ref[...], preferred_element_type=f32) + b1_ref[...]
    h = jnp.maximum(h, 0.0)
    o = jnp.dot(h, w2_ref[...], preferred_element_type=f32) + b2_ref[...]

    bo = pltpu.roll(cco, LW - 8, 1)
    co = pltpu.roll(bco, 8, 1)
    out_ref[...] = (o * a + pltpu.roll(o, LW - 8, 1) * bo
                    + pltpu.roll(o, 8, 1) * co)


def _dense_stage(gathered, de, rot_flat, w1bd, b1t, w2bd, b2t, blk=1000):
    n_rows = gathered.shape[0]
    nb = n_rows // blk

    def qmap(j):
        return lambda i, j=j: (j * nb + i, 0)

    return pl.pallas_call(
        _dense_body,
        grid=(nb,),
        in_specs=[
            pl.BlockSpec((blk, LW), lambda i: (i, 0)),
            pl.BlockSpec((blk, DIST), qmap(0)),
            pl.BlockSpec((blk, DIST), qmap(1)),
            pl.BlockSpec((blk, DIST), qmap(2)),
            pl.BlockSpec((blk, DIST), qmap(3)),
            pl.BlockSpec((blk, 8), qmap(0)),
            pl.BlockSpec((blk, 8), qmap(1)),
            pl.BlockSpec((blk, 8), qmap(2)),
            pl.BlockSpec((blk, 8), qmap(3)),
            pl.BlockSpec((NB_E * 8, LW), lambda i: (0, 0)),
            pl.BlockSpec((NB_E * 8, LW), lambda i: (0, 0)),
            pl.BlockSpec((NB_E * 8, LW), lambda i: (0, 0)),
            pl.BlockSpec((1, LW), lambda i: (0, 0)),
            pl.BlockSpec((NB_E * DIST + LW, NB_E * HID), lambda i: (0, 0)),
            pl.BlockSpec((1, NB_E * HID), lambda i: (0, 0)),
            pl.BlockSpec((NB_E * HID, LW), lambda i: (0, 0)),
            pl.BlockSpec((1, LW), lambda i: (0, 0)),
        ],
        out_specs=pl.BlockSpec((blk, LW), lambda i: (i, 0)),
        out_shape=jax.ShapeDtypeStruct((n_rows, LW), jnp.float32),
    )(gathered, de, de, de, de, rot_flat, rot_flat, rot_flat, rot_flat,
      jnp.asarray(_PA), jnp.asarray(_PB), jnp.asarray(_PC),
      jnp.asarray(_ONES8).reshape(1, LW), w1bd, b1t, w2bd, b2t)


def _make_weights(W1, b1, W2, b2):
    yrow = lambda p, l: 24 + (p & 3) * 4 + (p >> 2) * 2 + l
    in_rows = ([16 + r for r in range(8)]
               + [yrow(p, 0) for p in range(8)]
               + [yrow(p, 1) for p in range(8)])
    mcol = lambda p, m: 8 + (p & 3) * 4 + (p >> 2) * 2 + m
    out_cols = (list(range(8)) + [mcol(p, 0) for p in range(8)]
                + [mcol(p, 1) for p in range(8)])
    out_cols = np.array(out_cols)

    w1bd = jnp.zeros((NB_E * DIST + LW, NB_E * HID), jnp.float32)
    w2bd = jnp.zeros((NB_E * HID, LW), jnp.float32)
    for e in range(NB_E):
        w1bd = w1bd.at[e * DIST:(e + 1) * DIST,
                       e * HID:(e + 1) * HID].set(W1[:DIST])
        w1bd = w1bd.at[NB_E * DIST + e * ROW:NB_E * DIST + (e + 1) * ROW,
                       e * HID:(e + 1) * HID].set(W1[np.array(in_rows)])
        w2bd = w2bd.at[e * HID:(e + 1) * HID,
                       e * ROW:(e + 1) * ROW].set(W2[:, out_cols])
    b1t = jnp.tile(b1, NB_E).reshape(1, NB_E * HID)
    b2t = jnp.tile(b2[out_cols], NB_E).reshape(1, LW)
    return w1bd, b1t, w2bd, b2t


def _sc_scatter(msgs, col, n_edges, n_nodes):
    q = n_edges // NB_E
    half = n_nodes // 2
    acc_rows = 51200
    rows_per_tec = q // 16
    n_chunks = rows_per_tec // CR
    zero_rows_per_tec = acc_rows // 16
    drain_a = 3128
    drain_b = half - 15 * drain_a

    @functools.partial(
        pl.kernel,
        mesh=plsc.VectorSubcoreMesh(**_MESH),
        compiler_params=pltpu.CompilerParams(use_tc_tiling_on_sc=False),
        out_type=jax.ShapeDtypeStruct((n_nodes, ROW), jnp.float32),
        scratch_types=[
            pltpu.VMEM((2, CHE), jnp.int32),
            pltpu.VMEM((NBURST, IDXB), jnp.int32),
            pltpu.VMEM((2 * CHE, ROW), jnp.float32),
            pltpu.VMEM_SHARED((acc_rows, ROW), jnp.float32),
            pltpu.SemaphoreType.DMA,
            pltpu.SemaphoreType.DMA,
            pltpu.SemaphoreType.DMA,
        ],
    )
    def sk(msg_hbm, col_hbm, out_hbm, col_v, loc_v, msg_v, acc,
           lsem0, lsem1, ssem):
        c = lax.axis_index("c")
        t = lax.axis_index("s")
        node_base = c * half

        zeros16 = jnp.zeros((16,), jnp.float32)

        def zbody(i, carry):
            msg_v[i // 2, pl.ds((i % 2) * 8, 16)] = zeros16
            return carry
        lax.fori_loop(0, CHE * 2, zbody, 0)
        for qq in range(zero_rows_per_tec // CHE):
            pltpu.sync_copy(
                msg_v.at[pl.ds(0, CHE)],
                acc.at[pl.ds(t * zero_rows_per_tec + qq * CHE, CHE)])
        plsc.subcore_barrier()

        def row0_of(ci):
            return pl.multiple_of(t * rows_per_tec + ci * CR, CR)

        def issue_loads(ci, buf, lsem):
            row0 = row0_of(ci)
            for j in range(NB_E):
                pltpu.async_copy(col_hbm.at[pl.ds(j * q + row0, CR)],
                                 col_v.at[buf, pl.ds(j * CR, CR)], lsem)
                pltpu.async_copy(
                    msg_hbm.at[pl.ds(row0, CR), pl.ds(j * ROW, ROW)],
                    msg_v.at[pl.ds(buf * CHE + j * CR, CR)], lsem)

        def wait_loads(ci, buf, lsem):
            row0 = row0_of(ci)
            for j in range(NB_E):
                pltpu.make_async_copy(
                    col_hbm.at[pl.ds(j * q + row0, CR)],
                    col_v.at[buf, pl.ds(j * CR, CR)], lsem).wait()
                pltpu.make_async_copy(
                    msg_hbm.at[pl.ds(row0, CR), pl.ds(j * ROW, ROW)],
                    msg_v.at[pl.ds(buf * CHE + j * CR, CR)], lsem).wait()

        def process(buf):
            for b in range(NBURST):
                for gs in range(IDXB // 16):
                    v = col_v[buf, pl.ds(b * IDXB + gs * 16, 16)]
                    li = v - node_base
                    ok = (li >= 0) & (li < half)
                    loc_v[b, pl.ds(gs * 16, 16)] = jnp.where(ok, li, half)
            handles = []
            for b in range(NBURST):
                handles.append(pltpu.async_copy(
                    msg_v.at[pl.ds(buf * CHE + b * IDXB, IDXB)],
                    acc.at[loc_v.at[b]], ssem, add=True))
            for h in handles:
                h.wait()

        issue_loads(0, 0, lsem0)

        def body(i, carry):
            ci0 = i * 2
            wait_loads(ci0, 0, lsem0)

            @pl.when(ci0 + 1 < n_chunks)
            def _():
                issue_loads(ci0 + 1, 1, lsem1)
            process(0)

            @pl.when(ci0 + 1 < n_chunks)
            def _():
                wait_loads(ci0 + 1, 1, lsem1)

                @pl.when(ci0 + 2 < n_chunks)
                def _():
                    issue_loads(ci0 + 2, 0, lsem0)
                process(1)
            return carry

        lax.fori_loop(0, (n_chunks + 1) // 2, body, 0)
        plsc.subcore_barrier()
        off = pl.multiple_of(t * drain_a, 8)

        @pl.when(t < 15)
        def _():
            pltpu.sync_copy(
                acc.at[pl.ds(off, drain_a)],
                out_hbm.at[pl.ds(pl.multiple_of(node_base + off, 8),
                                 drain_a)])

        @pl.when(t == 15)
        def _():
            pltpu.sync_copy(
                acc.at[pl.ds(15 * drain_a, drain_b)],
                out_hbm.at[pl.ds(pl.multiple_of(node_base + 15 * drain_a, 8),
                                 drain_b)])

    return sk(msgs, col)


def kernel(x_scalar, x_rot, edge_index, distance_embedding, rot, W1, b1, W2,
           b2):
    n_nodes = x_rot.shape[0]
    n_edges = edge_index.shape[1]

    row = edge_index[0].astype(jnp.int32)
    col = edge_index[1].astype(jnp.int32)

    xr = x_rot.reshape(n_nodes, NUM_REP, L_MAX, 2)
    xr_m = jnp.transpose(xr, (0, 3, 2, 1)).reshape(n_nodes, 16)
    table = jnp.concatenate([x_scalar, xr_m], axis=1)

    w1bd, b1t, w2bd, b2t = _make_weights(W1, b1, W2, b2)

    gathered = _sc_gather(table, row, n_edges)
    msgs = _dense_stage(gathered, distance_embedding,
                        rot.reshape(n_edges, 8), w1bd, b1t, w2bd, b2t)
    acc = _sc_scatter(msgs, col, n_edges, n_nodes)

    mess_scalar = acc[:, :N_SCALARS]
    zm = acc[:, 8:24].reshape(n_nodes, 2, 2, NUM_REP)
    mess_rot = jnp.transpose(zm, (0, 3, 2, 1)).reshape(
        n_nodes, NUM_REP, L_MAX * 2)
    return (mess_scalar, mess_rot)


# speedup vs baseline: 4.6584x; 1.0432x over previous
import functools

import jax
import jax.numpy as jnp
import numpy as np
from jax import lax
from jax.experimental import pallas as pl
from jax.experimental.pallas import tpu as pltpu
from jax.experimental.pallas import tpu_sc as plsc

N_SCALARS = 8
NUM_REP = 4
L_MAX = 2
ROW = 24
NB_E = 4
LW = ROW * NB_E
DIST = 16
HID = 72

_MESH = dict(core_axis_name="c", subcore_axis_name="s", num_cores=2,
             num_subcores=16)
NW = 32
IDXB = 80
CR = 200
CHE = CR * NB_E
NBURST = CHE // IDXB


def _sc_gather(table, idx, n_edges):
    q = n_edges // NB_E
    rows_per_w = q // NW
    n_chunks = rows_per_w // CR

    @functools.partial(
        pl.kernel,
        mesh=plsc.VectorSubcoreMesh(**_MESH),
        compiler_params=pltpu.CompilerParams(use_tc_tiling_on_sc=False),
        out_type=jax.ShapeDtypeStruct((q, LW), jnp.float32),
        scratch_types=[
            pltpu.VMEM((CHE,), jnp.int32),
            pltpu.VMEM((CHE, ROW), jnp.float32),
            pltpu.SemaphoreType.DMA,
        ],
    )
    def gk(table_hbm, idx_hbm, out_hbm, idx_v, rows_v, sem):
        wid = lax.axis_index("s") * 2 + lax.axis_index("c")
        base = wid * rows_per_w

        def body(i, carry):
            row0 = pl.multiple_of(base + i * CR, CR)
            for j in range(NB_E):
                pltpu.sync_copy(idx_hbm.at[pl.ds(j * q + row0, CR)],
                                idx_v.at[pl.ds(j * CR, CR)])
            handles = []
            for b in range(NBURST):
                handles.append(pltpu.async_copy(
                    table_hbm.at[idx_v.at[pl.ds(b * IDXB, IDXB)]],
                    rows_v.at[pl.ds(b * IDXB, IDXB)], sem))
            for h in handles:
                h.wait()
            for j in range(NB_E):
                pltpu.sync_copy(
                    rows_v.at[pl.ds(j * CR, CR)],
                    out_hbm.at[pl.ds(row0, CR), pl.ds(j * ROW, ROW)])
            return carry

        lax.fori_loop(0, n_chunks, body, 0)

    return gk(table, idx)


def _make_consts():
    pa = np.zeros((32, LW), np.float32)
    pb = np.zeros((32, LW), np.float32)
    pc = np.zeros((32, LW), np.float32)
    for e in range(NB_E):
        for p in range(8):
            k = p >> 2
            pa[e * 8 + k * 4 + 0, e * ROW + 8 + p] = 1.0
            pa[e * 8 + k * 4 + 3, e * ROW + 16 + p] = 1.0
            pb[e * 8 + k * 4 + 1, e * ROW + 8 + p] = 1.0
            pc[e * 8 + k * 4 + 2, e * ROW + 16 + p] = 1.0
    ones8 = np.zeros((LW,), np.float32)
    for e in range(NB_E):
        ones8[e * ROW:e * ROW + 8] = 1.0
    return pa, pb, pc, ones8


_PA, _PB, _PC, _ONES8 = _make_consts()


def _dense_body(g_ref, de0_ref, de1_ref, de2_ref, de3_ref,
                rot0_ref, rot1_ref, rot2_ref, rot3_ref,
                pa_ref, pb_ref, pc_ref, ones_ref,
                w1_ref, b1_ref, w2_ref, b2_ref, out_ref):
    f32 = jnp.float32
    rot_b = jnp.concatenate(
        [rot0_ref[...], rot1_ref[...], rot2_ref[...], rot3_ref[...]],
        axis=1)
    a = jnp.dot(rot_b, pa_ref[...], preferred_element_type=f32) + ones_ref[...]
    bco = jnp.dot(rot_b, pb_ref[...], preferred_element_type=f32)
    cco = jnp.dot(rot_b, pc_ref[...], preferred_element_type=f32)

    g = g_ref[...]
    u = (g * a + pltpu.roll(g, LW - 8, 1) * bco
         + pltpu.roll(g, 8, 1) * cco)

    x_in = jnp.concatenate(
        [de0_ref[...], de1_ref[...], de2_ref[...], de3_ref[...], u],
        axis=1)
    h = jnp.dot(x_in, w1_ref[...], preferred_element_type=f32) + b1_ref[...]
    h = jnp.maximum(h, 0.0)
    o = jnp.dot(h, w2_ref[...], preferred_element_type=f32) + b2_ref[...]

    bo = pltpu.roll(cco, LW - 8, 1)
    co = pltpu.roll(bco, 8, 1)
    out_ref[...] = (o * a + pltpu.roll(o, LW - 8, 1) * bo
                    + pltpu.roll(o, 8, 1) * co)


def _dense_stage(gathered, de, rot_flat, w1bd, b1t, w2bd, b2t, blk=2000):
    n_rows = gathered.shape[0]
    nb = n_rows // blk

    def qmap(j):
        return lambda i, j=j: (j * nb + i, 0)

    return pl.pallas_call(
        _dense_body,
        grid=(nb,),
        in_specs=[
            pl.BlockSpec((blk, LW), lambda i: (i, 0)),
            pl.BlockSpec((blk, DIST), qmap(0)),
            pl.BlockSpec((blk, DIST), qmap(1)),
            pl.BlockSpec((blk, DIST), qmap(2)),
            pl.BlockSpec((blk, DIST), qmap(3)),
            pl.BlockSpec((blk, 8), qmap(0)),
            pl.BlockSpec((blk, 8), qmap(1)),
            pl.BlockSpec((blk, 8), qmap(2)),
            pl.BlockSpec((blk, 8), qmap(3)),
            pl.BlockSpec((NB_E * 8, LW), lambda i: (0, 0)),
            pl.BlockSpec((NB_E * 8, LW), lambda i: (0, 0)),
            pl.BlockSpec((NB_E * 8, LW), lambda i: (0, 0)),
            pl.BlockSpec((1, LW), lambda i: (0, 0)),
            pl.BlockSpec((NB_E * DIST + LW, NB_E * HID), lambda i: (0, 0)),
            pl.BlockSpec((1, NB_E * HID), lambda i: (0, 0)),
            pl.BlockSpec((NB_E * HID, LW), lambda i: (0, 0)),
            pl.BlockSpec((1, LW), lambda i: (0, 0)),
        ],
        out_specs=pl.BlockSpec((blk, LW), lambda i: (i, 0)),
        out_shape=jax.ShapeDtypeStruct((n_rows, LW), jnp.float32),
    )(gathered, de, de, de, de, rot_flat, rot_flat, rot_flat, rot_flat,
      jnp.asarray(_PA), jnp.asarray(_PB), jnp.asarray(_PC),
      jnp.asarray(_ONES8).reshape(1, LW), w1bd, b1t, w2bd, b2t)


def _make_weights(W1, b1, W2, b2):
    yrow = lambda p, l: 24 + (p & 3) * 4 + (p >> 2) * 2 + l
    in_rows = ([16 + r for r in range(8)]
               + [yrow(p, 0) for p in range(8)]
               + [yrow(p, 1) for p in range(8)])
    mcol = lambda p, m: 8 + (p & 3) * 4 + (p >> 2) * 2 + m
    out_cols = (list(range(8)) + [mcol(p, 0) for p in range(8)]
                + [mcol(p, 1) for p in range(8)])
    out_cols = np.array(out_cols)

    w1bd = jnp.zeros((NB_E * DIST + LW, NB_E * HID), jnp.float32)
    w2bd = jnp.zeros((NB_E * HID, LW), jnp.float32)
    for e in range(NB_E):
        w1bd = w1bd.at[e * DIST:(e + 1) * DIST,
                       e * HID:(e + 1) * HID].set(W1[:DIST])
        w1bd = w1bd.at[NB_E * DIST + e * ROW:NB_E * DIST + (e + 1) * ROW,
                       e * HID:(e + 1) * HID].set(W1[np.array(in_rows)])
        w2bd = w2bd.at[e * HID:(e + 1) * HID,
                       e * ROW:(e + 1) * ROW].set(W2[:, out_cols])
    b1t = jnp.tile(b1, NB_E).reshape(1, NB_E * HID)
    b2t = jnp.tile(b2[out_cols], NB_E).reshape(1, LW)
    return w1bd, b1t, w2bd, b2t


def _sc_scatter(msgs, col, n_edges, n_nodes):
    q = n_edges // NB_E
    half = n_nodes // 2
    acc_rows = 51200
    rows_per_tec = q // 16
    n_chunks = rows_per_tec // CR
    zero_rows_per_tec = acc_rows // 16
    drain_a = 3128
    drain_b = half - 15 * drain_a

    @functools.partial(
        pl.kernel,
        mesh=plsc.VectorSubcoreMesh(**_MESH),
        compiler_params=pltpu.CompilerParams(use_tc_tiling_on_sc=False),
        out_type=jax.ShapeDtypeStruct((n_nodes, ROW), jnp.float32),
        scratch_types=[
            pltpu.VMEM((2, CHE), jnp.int32),
            pltpu.VMEM((NBURST, IDXB), jnp.int32),
            pltpu.VMEM((2 * CHE, ROW), jnp.float32),
            pltpu.VMEM_SHARED((acc_rows, ROW), jnp.float32),
            pltpu.SemaphoreType.DMA,
            pltpu.SemaphoreType.DMA,
            pltpu.SemaphoreType.DMA,
        ],
    )
    def sk(msg_hbm, col_hbm, out_hbm, col_v, loc_v, msg_v, acc,
           lsem0, lsem1, ssem):
        c = lax.axis_index("c")
        t = lax.axis_index("s")
        node_base = c * half

        zeros16 = jnp.zeros((16,), jnp.float32)

        def zbody(i, carry):
            msg_v[i // 2, pl.ds((i % 2) * 8, 16)] = zeros16
            return carry
        lax.fori_loop(0, CHE * 2, zbody, 0)
        for qq in range(zero_rows_per_tec // CHE):
            pltpu.sync_copy(
                msg_v.at[pl.ds(0, CHE)],
                acc.at[pl.ds(t * zero_rows_per_tec + qq * CHE, CHE)])
        plsc.subcore_barrier()

        def row0_of(ci):
            return pl.multiple_of(t * rows_per_tec + ci * CR, CR)

        def issue_loads(ci, buf, lsem):
            row0 = row0_of(ci)
            for j in range(NB_E):
                pltpu.async_copy(col_hbm.at[pl.ds(j * q + row0, CR)],
                                 col_v.at[buf, pl.ds(j * CR, CR)], lsem)
                pltpu.async_copy(
                    msg_hbm.at[pl.ds(row0, CR), pl.ds(j * ROW, ROW)],
                    msg_v.at[pl.ds(buf * CHE + j * CR, CR)], lsem)

        def wait_loads(ci, buf, lsem):
            row0 = row0_of(ci)
            for j in range(NB_E):
                pltpu.make_async_copy(
                    col_hbm.at[pl.ds(j * q + row0, CR)],
                    col_v.at[buf, pl.ds(j * CR, CR)], lsem).wait()
                pltpu.make_async_copy(
                    msg_hbm.at[pl.ds(row0, CR), pl.ds(j * ROW, ROW)],
                    msg_v.at[pl.ds(buf * CHE + j * CR, CR)], lsem).wait()

        def process(buf):
            for b in range(NBURST):
                for gs in range(IDXB // 16):
                    v = col_v[buf, pl.ds(b * IDXB + gs * 16, 16)]
                    li = v - node_base
                    ok = (li >= 0) & (li < half)
                    loc_v[b, pl.ds(gs * 16, 16)] = jnp.where(ok, li, half)
            handles = []
            for b in range(NBURST):
                handles.append(pltpu.async_copy(
                    msg_v.at[pl.ds(buf * CHE + b * IDXB, IDXB)],
                    acc.at[loc_v.at[b]], ssem, add=True))
            for h in handles:
                h.wait()

        issue_loads(0, 0, lsem0)

        def body(i, carry):
            ci0 = i * 2
            wait_loads(ci0, 0, lsem0)

            @pl.when(ci0 + 1 < n_chunks)
            def _():
                issue_loads(ci0 + 1, 1, lsem1)
            process(0)

            @pl.when(ci0 + 1 < n_chunks)
            def _():
                wait_loads(ci0 + 1, 1, lsem1)

                @pl.when(ci0 + 2 < n_chunks)
                def _():
                    issue_loads(ci0 + 2, 0, lsem0)
                process(1)
            return carry

        lax.fori_loop(0, (n_chunks + 1) // 2, body, 0)
        plsc.subcore_barrier()
        off = pl.multiple_of(t * drain_a, 8)

        @pl.when(t < 15)
        def _():
            pltpu.sync_copy(
                acc.at[pl.ds(off, drain_a)],
                out_hbm.at[pl.ds(pl.multiple_of(node_base + off, 8),
                                 drain_a)])

        @pl.when(t == 15)
        def _():
            pltpu.sync_copy(
                acc.at[pl.ds(15 * drain_a, drain_b)],
                out_hbm.at[pl.ds(pl.multiple_of(node_base + 15 * drain_a, 8),
                                 drain_b)])

    return sk(msgs, col)


def kernel(x_scalar, x_rot, edge_index, distance_embedding, rot, W1, b1, W2,
           b2):
    n_nodes = x_rot.shape[0]
    n_edges = edge_index.shape[1]

    row = edge_index[0].astype(jnp.int32)
    col = edge_index[1].astype(jnp.int32)

    xr = x_rot.reshape(n_nodes, NUM_REP, L_MAX, 2)
    xr_m = jnp.transpose(xr, (0, 3, 2, 1)).reshape(n_nodes, 16)
    table = jnp.concatenate([x_scalar, xr_m], axis=1)

    w1bd, b1t, w2bd, b2t = _make_weights(W1, b1, W2, b2)

    gathered = _sc_gather(table, row, n_edges)
    msgs = _dense_stage(gathered, distance_embedding,
                        rot.reshape(n_edges, 8), w1bd, b1t, w2bd, b2t)
    acc = _sc_scatter(msgs, col, n_edges, n_nodes)

    mess_scalar = acc[:, :N_SCALARS]
    zm = acc[:, 8:24].reshape(n_nodes, 2, 2, NUM_REP)
    mess_rot = jnp.transpose(zm, (0, 3, 2, 1)).reshape(
        n_nodes, NUM_REP, L_MAX * 2)
    return (mess_scalar, mess_rot)


# speedup vs baseline: 4.7379x; 1.0171x over previous
import functools

import jax
import jax.numpy as jnp
import numpy as np
from jax import lax
from jax.experimental import pallas as pl
from jax.experimental.pallas import tpu as pltpu
from jax.experimental.pallas import tpu_sc as plsc

N_SCALARS = 8
NUM_REP = 4
L_MAX = 2
ROW = 24
NB_E = 4
LW = ROW * NB_E
DIST = 16
HID = 72

_MESH = dict(core_axis_name="c", subcore_axis_name="s", num_cores=2,
             num_subcores=16)
NW = 32
IDXB = 80
CR = 200
CHE = CR * NB_E
NBURST = CHE // IDXB


def _sc_gather(table, idx, n_edges):
    q = n_edges // NB_E
    rows_per_w = q // NW
    n_chunks = rows_per_w // CR

    @functools.partial(
        pl.kernel,
        mesh=plsc.VectorSubcoreMesh(**_MESH),
        compiler_params=pltpu.CompilerParams(use_tc_tiling_on_sc=False),
        out_type=jax.ShapeDtypeStruct((q, LW), jnp.float32),
        scratch_types=[
            pltpu.VMEM((CHE,), jnp.int32),
            pltpu.VMEM((CHE, ROW), jnp.float32),
            pltpu.SemaphoreType.DMA,
        ],
    )
    def gk(table_hbm, idx_hbm, out_hbm, idx_v, rows_v, sem):
        wid = lax.axis_index("s") * 2 + lax.axis_index("c")
        base = wid * rows_per_w

        def body(i, carry):
            row0 = pl.multiple_of(base + i * CR, CR)
            for j in range(NB_E):
                pltpu.sync_copy(idx_hbm.at[pl.ds(j * q + row0, CR)],
                                idx_v.at[pl.ds(j * CR, CR)])
            handles = []
            for b in range(NBURST):
                handles.append(pltpu.async_copy(
                    table_hbm.at[idx_v.at[pl.ds(b * IDXB, IDXB)]],
                    rows_v.at[pl.ds(b * IDXB, IDXB)], sem))
            for h in handles:
                h.wait()
            for j in range(NB_E):
                pltpu.sync_copy(
                    rows_v.at[pl.ds(j * CR, CR)],
                    out_hbm.at[pl.ds(row0, CR), pl.ds(j * ROW, ROW)])
            return carry

        lax.fori_loop(0, n_chunks, body, 0)

    return gk(table, idx)


def _make_consts():
    pa = np.zeros((32, LW), np.float32)
    pb = np.zeros((32, LW), np.float32)
    pc = np.zeros((32, LW), np.float32)
    for e in range(NB_E):
        for p in range(8):
            k = p >> 2
            pa[e * 8 + k * 4 + 0, e * ROW + 8 + p] = 1.0
            pa[e * 8 + k * 4 + 3, e * ROW + 16 + p] = 1.0
            pb[e * 8 + k * 4 + 1, e * ROW + 8 + p] = 1.0
            pc[e * 8 + k * 4 + 2, e * ROW + 16 + p] = 1.0
    ones8 = np.zeros((LW,), np.float32)
    for e in range(NB_E):
        ones8[e * ROW:e * ROW + 8] = 1.0
    return pa, pb, pc, ones8


_PA, _PB, _PC, _ONES8 = _make_consts()


def _dense_body(g_ref, de_ref, rot_ref,
                pa_ref, pb_ref, pc_ref, ones_ref,
                w1_ref, b1_ref, w2_ref, b2_ref, out_ref):
    f32 = jnp.float32
    rot_b = rot_ref[...]
    a = jnp.dot(rot_b, pa_ref[...], preferred_element_type=f32) + ones_ref[...]
    bco = jnp.dot(rot_b, pb_ref[...], preferred_element_type=f32)
    cco = jnp.dot(rot_b, pc_ref[...], preferred_element_type=f32)

    g = g_ref[...]
    u = (g * a + pltpu.roll(g, LW - 8, 1) * bco
         + pltpu.roll(g, 8, 1) * cco)

    x_in = jnp.concatenate([de_ref[...], u], axis=1)
    h = jnp.dot(x_in, w1_ref[...], preferred_element_type=f32) + b1_ref[...]
    h = jnp.maximum(h, 0.0)
    o = jnp.dot(h, w2_ref[...], preferred_element_type=f32) + b2_ref[...]

    bo = pltpu.roll(cco, LW - 8, 1)
    co = pltpu.roll(bco, 8, 1)
    out_ref[...] = (o * a + pltpu.roll(o, LW - 8, 1) * bo
                    + pltpu.roll(o, 8, 1) * co)


def _dense_stage(gathered, de, rot_flat, w1bd, b1t, w2bd, b2t, blk=2000):
    n_rows = gathered.shape[0]
    q = n_rows
    de_p = jnp.concatenate([de[j * q:(j + 1) * q] for j in range(NB_E)],
                           axis=1)
    rot_p = jnp.concatenate(
        [rot_flat[j * q:(j + 1) * q] for j in range(NB_E)], axis=1)
    return pl.pallas_call(
        _dense_body,
        grid=(n_rows // blk,),
        in_specs=[
            pl.BlockSpec((blk, LW), lambda i: (i, 0)),
            pl.BlockSpec((blk, NB_E * DIST), lambda i: (i, 0)),
            pl.BlockSpec((blk, NB_E * 8), lambda i: (i, 0)),
            pl.BlockSpec((NB_E * 8, LW), lambda i: (0, 0)),
            pl.BlockSpec((NB_E * 8, LW), lambda i: (0, 0)),
            pl.BlockSpec((NB_E * 8, LW), lambda i: (0, 0)),
            pl.BlockSpec((1, LW), lambda i: (0, 0)),
            pl.BlockSpec((NB_E * DIST + LW, NB_E * HID), lambda i: (0, 0)),
            pl.BlockSpec((1, NB_E * HID), lambda i: (0, 0)),
            pl.BlockSpec((NB_E * HID, LW), lambda i: (0, 0)),
            pl.BlockSpec((1, LW), lambda i: (0, 0)),
        ],
        out_specs=pl.BlockSpec((blk, LW), lambda i: (i, 0)),
        out_shape=jax.ShapeDtypeStruct((n_rows, LW), jnp.float32),
    )(gathered, de_p, rot_p,
      jnp.asarray(_PA), jnp.asarray(_PB), jnp.asarray(_PC),
      jnp.asarray(_ONES8).reshape(1, LW), w1bd, b1t, w2bd, b2t)


def _make_weights(W1, b1, W2, b2):
    yrow = lambda p, l: 24 + (p & 3) * 4 + (p >> 2) * 2 + l
    in_rows = ([16 + r for r in range(8)]
               + [yrow(p, 0) for p in range(8)]
               + [yrow(p, 1) for p in range(8)])
    mcol = lambda p, m: 8 + (p & 3) * 4 + (p >> 2) * 2 + m
    out_cols = (list(range(8)) + [mcol(p, 0) for p in range(8)]
                + [mcol(p, 1) for p in range(8)])
    out_cols = np.array(out_cols)

    w1bd = jnp.zeros((NB_E * DIST + LW, NB_E * HID), jnp.float32)
    w2bd = jnp.zeros((NB_E * HID, LW), jnp.float32)
    for e in range(NB_E):
        w1bd = w1bd.at[e * DIST:(e + 1) * DIST,
                       e * HID:(e + 1) * HID].set(W1[:DIST])
        w1bd = w1bd.at[NB_E * DIST + e * ROW:NB_E * DIST + (e + 1) * ROW,
                       e * HID:(e + 1) * HID].set(W1[np.array(in_rows)])
        w2bd = w2bd.at[e * HID:(e + 1) * HID,
                       e * ROW:(e + 1) * ROW].set(W2[:, out_cols])
    b1t = jnp.tile(b1, NB_E).reshape(1, NB_E * HID)
    b2t = jnp.tile(b2[out_cols], NB_E).reshape(1, LW)
    return w1bd, b1t, w2bd, b2t


def _sc_scatter(msgs, col, n_edges, n_nodes):
    q = n_edges // NB_E
    half = n_nodes // 2
    acc_rows = 51200
    rows_per_tec = q // 16
    n_chunks = rows_per_tec // CR
    zero_rows_per_tec = acc_rows // 16
    drain_a = 3128
    drain_b = half - 15 * drain_a

    @functools.partial(
        pl.kernel,
        mesh=plsc.VectorSubcoreMesh(**_MESH),
        compiler_params=pltpu.CompilerParams(use_tc_tiling_on_sc=False),
        out_type=jax.ShapeDtypeStruct((n_nodes, ROW), jnp.float32),
        scratch_types=[
            pltpu.VMEM((2, CHE), jnp.int32),
            pltpu.VMEM((NBURST, IDXB), jnp.int32),
            pltpu.VMEM((2 * CHE, ROW), jnp.float32),
            pltpu.VMEM_SHARED((acc_rows, ROW), jnp.float32),
            pltpu.SemaphoreType.DMA,
            pltpu.SemaphoreType.DMA,
            pltpu.SemaphoreType.DMA,
        ],
    )
    def sk(msg_hbm, col_hbm, out_hbm, col_v, loc_v, msg_v, acc,
           lsem0, lsem1, ssem):
        c = lax.axis_index("c")
        t = lax.axis_index("s")
        node_base = c * half

        zeros16 = jnp.zeros((16,), jnp.float32)

        def zbody(i, carry):
            msg_v[i // 2, pl.ds((i % 2) * 8, 16)] = zeros16
            return carry
        lax.fori_loop(0, CHE * 2, zbody, 0)
        for qq in range(zero_rows_per_tec // CHE):
            pltpu.sync_copy(
                msg_v.at[pl.ds(0, CHE)],
                acc.at[pl.ds(t * zero_rows_per_tec + qq * CHE, CHE)])
        plsc.subcore_barrier()

        def row0_of(ci):
            return pl.multiple_of(t * rows_per_tec + ci * CR, CR)

        def issue_loads(ci, buf, lsem):
            row0 = row0_of(ci)
            for j in range(NB_E):
                pltpu.async_copy(col_hbm.at[pl.ds(j * q + row0, CR)],
                                 col_v.at[buf, pl.ds(j * CR, CR)], lsem)
                pltpu.async_copy(
                    msg_hbm.at[pl.ds(row0, CR), pl.ds(j * ROW, ROW)],
                    msg_v.at[pl.ds(buf * CHE + j * CR, CR)], lsem)

        def wait_loads(ci, buf, lsem):
            row0 = row0_of(ci)
            for j in range(NB_E):
                pltpu.make_async_copy(
                    col_hbm.at[pl.ds(j * q + row0, CR)],
                    col_v.at[buf, pl.ds(j * CR, CR)], lsem).wait()
                pltpu.make_async_copy(
                    msg_hbm.at[pl.ds(row0, CR), pl.ds(j * ROW, ROW)],
                    msg_v.at[pl.ds(buf * CHE + j * CR, CR)], lsem).wait()

        def process(buf):
            for b in range(NBURST):
                for gs in range(IDXB // 16):
                    v = col_v[buf, pl.ds(b * IDXB + gs * 16, 16)]
                    li = v - node_base
                    ok = (li >= 0) & (li < half)
                    loc_v[b, pl.ds(gs * 16, 16)] = jnp.where(ok, li, half)
            handles = []
            for b in range(NBURST):
                handles.append(pltpu.async_copy(
                    msg_v.at[pl.ds(buf * CHE + b * IDXB, IDXB)],
                    acc.at[loc_v.at[b]], ssem, add=True))
            for h in handles:
                h.wait()

        issue_loads(0, 0, lsem0)

        def body(i, carry):
            ci0 = i * 2
            wait_loads(ci0, 0, lsem0)

            @pl.when(ci0 + 1 < n_chunks)
            def _():
                issue_loads(ci0 + 1, 1, lsem1)
            process(0)

            @pl.when(ci0 + 1 < n_chunks)
            def _():
                wait_loads(ci0 + 1, 1, lsem1)

                @pl.when(ci0 + 2 < n_chunks)
                def _():
                    issue_loads(ci0 + 2, 0, lsem0)
                process(1)
            return carry

        lax.fori_loop(0, (n_chunks + 1) // 2, body, 0)
        plsc.subcore_barrier()
        off = pl.multiple_of(t * drain_a, 8)

        @pl.when(t < 15)
        def _():
            pltpu.sync_copy(
                acc.at[pl.ds(off, drain_a)],
                out_hbm.at[pl.ds(pl.multiple_of(node_base + off, 8),
                                 drain_a)])

        @pl.when(t == 15)
        def _():
            pltpu.sync_copy(
                acc.at[pl.ds(15 * drain_a, drain_b)],
                out_hbm.at[pl.ds(pl.multiple_of(node_base + 15 * drain_a, 8),
                                 drain_b)])

    return sk(msgs, col)


def kernel(x_scalar, x_rot, edge_index, distance_embedding, rot, W1, b1, W2,
           b2):
    n_nodes = x_rot.shape[0]
    n_edges = edge_index.shape[1]

    row = edge_index[0].astype(jnp.int32)
    col = edge_index[1].astype(jnp.int32)

    xr = x_rot.reshape(n_nodes, NUM_REP, L_MAX, 2)
    xr_m = jnp.transpose(xr, (0, 3, 2, 1)).reshape(n_nodes, 16)
    table = jnp.concatenate([x_scalar, xr_m], axis=1)

    w1bd, b1t, w2bd, b2t = _make_weights(W1, b1, W2, b2)

    gathered = _sc_gather(table, row, n_edges)
    msgs = _dense_stage(gathered, distance_embedding,
                        rot.reshape(n_edges, 8), w1bd, b1t, w2bd, b2t)
    acc = _sc_scatter(msgs, col, n_edges, n_nodes)

    mess_scalar = acc[:, :N_SCALARS]
    zm = acc[:, 8:24].reshape(n_nodes, 2, 2, NUM_REP)
    mess_rot = jnp.transpose(zm, (0, 3, 2, 1)).reshape(
        n_nodes, NUM_REP, L_MAX * 2)
    return (mess_scalar, mess_rot)


# speedup vs baseline: 4.7648x; 1.0057x over previous
import functools

import jax
import jax.numpy as jnp
import numpy as np
from jax import lax
from jax.experimental import pallas as pl
from jax.experimental.pallas import tpu as pltpu
from jax.experimental.pallas import tpu_sc as plsc

N_SCALARS = 8
NUM_REP = 4
L_MAX = 2
ROW = 24
NB_E = 4
LW = ROW * NB_E
DIST = 16
HID = 72

_MESH = dict(core_axis_name="c", subcore_axis_name="s", num_cores=2,
             num_subcores=16)
NW = 32
IDXB = 80
CR = 200
CHE = CR * NB_E
NBURST = CHE // IDXB


def _sc_gather(table, idx, n_edges):
    q = n_edges // NB_E
    rows_per_w = q // NW
    n_chunks = rows_per_w // CR

    @functools.partial(
        pl.kernel,
        mesh=plsc.VectorSubcoreMesh(**_MESH),
        compiler_params=pltpu.CompilerParams(use_tc_tiling_on_sc=False),
        out_type=jax.ShapeDtypeStruct((q, LW), jnp.float32),
        scratch_types=[
            pltpu.VMEM((CHE,), jnp.int32),
            pltpu.VMEM((CHE, ROW), jnp.float32),
            pltpu.SemaphoreType.DMA,
        ],
    )
    def gk(table_hbm, idx_hbm, out_hbm, idx_v, rows_v, sem):
        wid = lax.axis_index("s") * 2 + lax.axis_index("c")
        base = wid * rows_per_w

        def body(i, carry):
            row0 = pl.multiple_of(base + i * CR, CR)
            for j in range(NB_E):
                pltpu.sync_copy(idx_hbm.at[pl.ds(j * q + row0, CR)],
                                idx_v.at[pl.ds(j * CR, CR)])
            handles = []
            for b in range(NBURST):
                handles.append(pltpu.async_copy(
                    table_hbm.at[idx_v.at[pl.ds(b * IDXB, IDXB)]],
                    rows_v.at[pl.ds(b * IDXB, IDXB)], sem))
            for h in handles:
                h.wait()
            for j in range(NB_E):
                pltpu.sync_copy(
                    rows_v.at[pl.ds(j * CR, CR)],
                    out_hbm.at[pl.ds(row0, CR), pl.ds(j * ROW, ROW)])
            return carry

        lax.fori_loop(0, n_chunks, body, 0)

    return gk(table, idx)


def _make_consts():
    pa = np.zeros((32, LW), np.float32)
    pb = np.zeros((32, LW), np.float32)
    pc = np.zeros((32, LW), np.float32)
    for e in range(NB_E):
        for p in range(8):
            k = p >> 2
            pa[e * 8 + k * 4 + 0, e * ROW + 8 + p] = 1.0
            pa[e * 8 + k * 4 + 3, e * ROW + 16 + p] = 1.0
            pb[e * 8 + k * 4 + 1, e * ROW + 8 + p] = 1.0
            pc[e * 8 + k * 4 + 2, e * ROW + 16 + p] = 1.0
    ones8 = np.zeros((LW,), np.float32)
    for e in range(NB_E):
        ones8[e * ROW:e * ROW + 8] = 1.0
    return pa, pb, pc, ones8


_PA, _PB, _PC, _ONES8 = _make_consts()


def _dense_body(g_ref, de_ref, rot_ref,
                pa_ref, pb_ref, pc_ref, ones_ref,
                w1_ref, b1_ref, w2_ref, b2_ref, out_ref):
    f32 = jnp.float32
    rot_b = rot_ref[...]
    a = jnp.dot(rot_b, pa_ref[...], preferred_element_type=f32) + ones_ref[...]
    bco = jnp.dot(rot_b, pb_ref[...], preferred_element_type=f32)
    cco = jnp.dot(rot_b, pc_ref[...], preferred_element_type=f32)

    g = g_ref[...]
    u = (g * a + pltpu.roll(g, LW - 8, 1) * bco
         + pltpu.roll(g, 8, 1) * cco)

    x_in = jnp.concatenate([de_ref[...], u], axis=1)
    h = jnp.dot(x_in, w1_ref[...], preferred_element_type=f32) + b1_ref[...]
    h = jnp.maximum(h, 0.0)
    o = jnp.dot(h, w2_ref[...], preferred_element_type=f32) + b2_ref[...]

    bo = pltpu.roll(cco, LW - 8, 1)
    co = pltpu.roll(bco, 8, 1)
    out_ref[...] = (o * a + pltpu.roll(o, LW - 8, 1) * bo
                    + pltpu.roll(o, 8, 1) * co)


def _dense_stage(gathered, de, rot_flat, w1bd, b1t, w2bd, b2t, blk=4000):
    n_rows = gathered.shape[0]
    q = n_rows
    de_p = jnp.concatenate([de[j * q:(j + 1) * q] for j in range(NB_E)],
                           axis=1)
    rot_p = jnp.concatenate(
        [rot_flat[j * q:(j + 1) * q] for j in range(NB_E)], axis=1)
    return pl.pallas_call(
        _dense_body,
        grid=(n_rows // blk,),
        in_specs=[
            pl.BlockSpec((blk, LW), lambda i: (i, 0)),
            pl.BlockSpec((blk, NB_E * DIST), lambda i: (i, 0)),
            pl.BlockSpec((blk, NB_E * 8), lambda i: (i, 0)),
            pl.BlockSpec((NB_E * 8, LW), lambda i: (0, 0)),
            pl.BlockSpec((NB_E * 8, LW), lambda i: (0, 0)),
            pl.BlockSpec((NB_E * 8, LW), lambda i: (0, 0)),
            pl.BlockSpec((1, LW), lambda i: (0, 0)),
            pl.BlockSpec((NB_E * DIST + LW, NB_E * HID), lambda i: (0, 0)),
            pl.BlockSpec((1, NB_E * HID), lambda i: (0, 0)),
            pl.BlockSpec((NB_E * HID, LW), lambda i: (0, 0)),
            pl.BlockSpec((1, LW), lambda i: (0, 0)),
        ],
        out_specs=pl.BlockSpec((blk, LW), lambda i: (i, 0)),
        out_shape=jax.ShapeDtypeStruct((n_rows, LW), jnp.float32),
    )(gathered, de_p, rot_p,
      jnp.asarray(_PA), jnp.asarray(_PB), jnp.asarray(_PC),
      jnp.asarray(_ONES8).reshape(1, LW), w1bd, b1t, w2bd, b2t)


def _make_weights(W1, b1, W2, b2):
    yrow = lambda p, l: 24 + (p & 3) * 4 + (p >> 2) * 2 + l
    in_rows = ([16 + r for r in range(8)]
               + [yrow(p, 0) for p in range(8)]
               + [yrow(p, 1) for p in range(8)])
    mcol = lambda p, m: 8 + (p & 3) * 4 + (p >> 2) * 2 + m
    out_cols = (list(range(8)) + [mcol(p, 0) for p in range(8)]
                + [mcol(p, 1) for p in range(8)])
    out_cols = np.array(out_cols)

    w1bd = jnp.zeros((NB_E * DIST + LW, NB_E * HID), jnp.float32)
    w2bd = jnp.zeros((NB_E * HID, LW), jnp.float32)
    for e in range(NB_E):
        w1bd = w1bd.at[e * DIST:(e + 1) * DIST,
                       e * HID:(e + 1) * HID].set(W1[:DIST])
        w1bd = w1bd.at[NB_E * DIST + e * ROW:NB_E * DIST + (e + 1) * ROW,
                       e * HID:(e + 1) * HID].set(W1[np.array(in_rows)])
        w2bd = w2bd.at[e * HID:(e + 1) * HID,
                       e * ROW:(e + 1) * ROW].set(W2[:, out_cols])
    b1t = jnp.tile(b1, NB_E).reshape(1, NB_E * HID)
    b2t = jnp.tile(b2[out_cols], NB_E).reshape(1, LW)
    return w1bd, b1t, w2bd, b2t


def _sc_scatter(msgs, col, n_edges, n_nodes):
    q = n_edges // NB_E
    half = n_nodes // 2
    acc_rows = 51200
    rows_per_tec = q // 16
    n_chunks = rows_per_tec // CR
    zero_rows_per_tec = acc_rows // 16
    drain_a = 3128
    drain_b = half - 15 * drain_a

    @functools.partial(
        pl.kernel,
        mesh=plsc.VectorSubcoreMesh(**_MESH),
        compiler_params=pltpu.CompilerParams(use_tc_tiling_on_sc=False),
        out_type=jax.ShapeDtypeStruct((n_nodes, ROW), jnp.float32),
        scratch_types=[
            pltpu.VMEM((2, CHE), jnp.int32),
            pltpu.VMEM((NBURST, IDXB), jnp.int32),
            pltpu.VMEM((2 * CHE, ROW), jnp.float32),
            pltpu.VMEM_SHARED((acc_rows, ROW), jnp.float32),
            pltpu.SemaphoreType.DMA,
            pltpu.SemaphoreType.DMA,
            pltpu.SemaphoreType.DMA,
        ],
    )
    def sk(msg_hbm, col_hbm, out_hbm, col_v, loc_v, msg_v, acc,
           lsem0, lsem1, ssem):
        c = lax.axis_index("c")
        t = lax.axis_index("s")
        node_base = c * half

        zeros16 = jnp.zeros((16,), jnp.float32)

        def zbody(i, carry):
            msg_v[i // 2, pl.ds((i % 2) * 8, 16)] = zeros16
            return carry
        lax.fori_loop(0, CHE * 2, zbody, 0)
        for qq in range(zero_rows_per_tec // CHE):
            pltpu.sync_copy(
                msg_v.at[pl.ds(0, CHE)],
                acc.at[pl.ds(t * zero_rows_per_tec + qq * CHE, CHE)])
        plsc.subcore_barrier()

        def row0_of(ci):
            return pl.multiple_of(t * rows_per_tec + ci * CR, CR)

        def issue_loads(ci, buf, lsem):
            row0 = row0_of(ci)
            for j in range(NB_E):
                pltpu.async_copy(col_hbm.at[pl.ds(j * q + row0, CR)],
                                 col_v.at[buf, pl.ds(j * CR, CR)], lsem)
                pltpu.async_copy(
                    msg_hbm.at[pl.ds(row0, CR), pl.ds(j * ROW, ROW)],
                    msg_v.at[pl.ds(buf * CHE + j * CR, CR)], lsem)

        def wait_loads(ci, buf, lsem):
            row0 = row0_of(ci)
            for j in range(NB_E):
                pltpu.make_async_copy(
                    col_hbm.at[pl.ds(j * q + row0, CR)],
                    col_v.at[buf, pl.ds(j * CR, CR)], lsem).wait()
                pltpu.make_async_copy(
                    msg_hbm.at[pl.ds(row0, CR), pl.ds(j * ROW, ROW)],
                    msg_v.at[pl.ds(buf * CHE + j * CR, CR)], lsem).wait()

        def process(buf):
            for b in range(NBURST):
                for gs in range(IDXB // 16):
                    v = col_v[buf, pl.ds(b * IDXB + gs * 16, 16)]
                    li = v - node_base
                    ok = (li >= 0) & (li < half)
                    loc_v[b, pl.ds(gs * 16, 16)] = jnp.where(ok, li, half)
            handles = []
            for b in range(NBURST):
                handles.append(pltpu.async_copy(
                    msg_v.at[pl.ds(buf * CHE + b * IDXB, IDXB)],
                    acc.at[loc_v.at[b]], ssem, add=True))
            for h in handles:
                h.wait()

        issue_loads(0, 0, lsem0)

        def body(i, carry):
            ci0 = i * 2
            wait_loads(ci0, 0, lsem0)

            @pl.when(ci0 + 1 < n_chunks)
            def _():
                issue_loads(ci0 + 1, 1, lsem1)
            process(0)

            @pl.when(ci0 + 1 < n_chunks)
            def _():
                wait_loads(ci0 + 1, 1, lsem1)

                @pl.when(ci0 + 2 < n_chunks)
                def _():
                    issue_loads(ci0 + 2, 0, lsem0)
                process(1)
            return carry

        lax.fori_loop(0, (n_chunks + 1) // 2, body, 0)
        plsc.subcore_barrier()
        off = pl.multiple_of(t * drain_a, 8)

        @pl.when(t < 15)
        def _():
            pltpu.sync_copy(
                acc.at[pl.ds(off, drain_a)],
                out_hbm.at[pl.ds(pl.multiple_of(node_base + off, 8),
                                 drain_a)])

        @pl.when(t == 15)
        def _():
            pltpu.sync_copy(
                acc.at[pl.ds(15 * drain_a, drain_b)],
                out_hbm.at[pl.ds(pl.multiple_of(node_base + 15 * drain_a, 8),
                                 drain_b)])

    return sk(msgs, col)


def kernel(x_scalar, x_rot, edge_index, distance_embedding, rot, W1, b1, W2,
           b2):
    n_nodes = x_rot.shape[0]
    n_edges = edge_index.shape[1]

    row = edge_index[0].astype(jnp.int32)
    col = edge_index[1].astype(jnp.int32)

    xr = x_rot.reshape(n_nodes, NUM_REP, L_MAX, 2)
    xr_m = jnp.transpose(xr, (0, 3, 2, 1)).reshape(n_nodes, 16)
    table = jnp.concatenate([x_scalar, xr_m], axis=1)

    w1bd, b1t, w2bd, b2t = _make_weights(W1, b1, W2, b2)

    gathered = _sc_gather(table, row, n_edges)
    msgs = _dense_stage(gathered, distance_embedding,
                        rot.reshape(n_edges, 8), w1bd, b1t, w2bd, b2t)
    acc = _sc_scatter(msgs, col, n_edges, n_nodes)

    mess_scalar = acc[:, :N_SCALARS]
    zm = acc[:, 8:24].reshape(n_nodes, 2, 2, NUM_REP)
    mess_rot = jnp.transpose(zm, (0, 3, 2, 1)).reshape(
        n_nodes, NUM_REP, L_MAX * 2)
    return (mess_scalar, mess_rot)
